# Initial kernel scaffold; baseline (speedup 1.0000x reference)
import jax
import jax.numpy as jnp
from jax.experimental import pallas as pl


def kernel(x, pos_encoding):
    raise NotImplementedError("write your pallas kernel here")


# speedup vs baseline: 1.0391x; 1.0391x over previous
import math

import jax
import jax.numpy as jnp
from jax.experimental import pallas as pl


def kernel(x, pos_encoding):
    b, l, d = x.shape
    scale = math.sqrt(d)
    pe = pos_encoding[:l]

    blk = 256
    while l % blk:
        blk //= 2

    def body(x_ref, pe_ref, o_ref):
        xv = x_ref[...]
        y = xv * scale + pe_ref[...][None, :, :]
        o_ref[...] = jnp.where(xv == 0.0, 0.0, y)

    return pl.pallas_call(
        body,
        grid=(l // blk,),
        in_specs=[
            pl.BlockSpec((b, blk, d), lambda i: (0, i, 0)),
            pl.BlockSpec((blk, d), lambda i: (i, 0)),
        ],
        out_specs=pl.BlockSpec((b, blk, d), lambda i: (0, i, 0)),
        out_shape=jax.ShapeDtypeStruct((b, l, d), x.dtype),
    )(x, pe)


# speedup vs baseline: 1.0538x; 1.0142x over previous
import math

import jax
import jax.numpy as jnp
from jax.experimental import pallas as pl


def kernel(x, pos_encoding):
    b, l, d = x.shape
    scale = math.sqrt(d)
    pe = pos_encoding[:l]

    blk = 512
    while l % blk:
        blk //= 2

    def body(x_ref, pe_ref, o_ref):
        xv = x_ref[...]
        y = xv * scale + pe_ref[...][None, :, :]
        o_ref[...] = jnp.where(xv == 0.0, 0.0, y)

    return pl.pallas_call(
        body,
        grid=(l // blk,),
        in_specs=[
            pl.BlockSpec((b, blk, d), lambda i: (0, i, 0)),
            pl.BlockSpec((blk, d), lambda i: (i, 0)),
        ],
        out_specs=pl.BlockSpec((b, blk, d), lambda i: (0, i, 0)),
        out_shape=jax.ShapeDtypeStruct((b, l, d), x.dtype),
    )(x, pe)
